# baseline (speedup 1.0000x reference)
import jax
import jax.numpy as jnp
from jax import lax
from jax.experimental import pallas as pl
from jax.experimental.pallas import tpu as pltpu
from jax.experimental.pallas import tpu_sc as plsc

B, L, D = 16384, 200, 32
NC, NU = 100000, 100000
DH = D // 2

NCORES, NSUB, LANES = 2, 16, 16
RPT = B // NSUB
CB = 4
NCHUNK = RPT // CB
IDX_N = CB * L
WIN = 128
NWIN_MAX = -(-(IDX_N + CB) // WIN)
CAP = NWIN_MAX * WIN
NPAD = 100096
SHARD = NPAD // NSUB


def _splat(v):
    return jnp.full((LANES,), 0, jnp.int32) + v


def _to_scalar(vec):
    return lax.reduce_max(vec, axes=(0,))


def _sc_body(ct_hbm, cc_hbm, ut_hbm, nu_hbm,
             ctw_lo, ctw_hi, utw_lo, utw_hi,
             out_lo, out_hi,
             ctraw0, utraw0, ccv0, ctraw1, utraw1, ccv1,
             nuv, ctc, utc, starts, ctrows, utrows, outv,
             spm_ct, spm_ut, semi0, semi1, semg):
    c = lax.axis_index("c")
    s = lax.axis_index("s")
    row0 = s * RPT
    lane = lax.iota(jnp.int32, LANES)
    lane0 = lane == 0

    shard = pl.ds(s * SHARD, SHARD)

    @pl.when(c == 0)
    def _stage_lo():
        pltpu.sync_copy(ctw_lo.at[shard], spm_ct.at[shard])
        pltpu.sync_copy(utw_lo.at[shard], spm_ut.at[shard])

    @pl.when(c == 1)
    def _stage_hi():
        pltpu.sync_copy(ctw_hi.at[shard], spm_ct.at[shard])
        pltpu.sync_copy(utw_hi.at[shard], spm_ut.at[shard])

    pltpu.sync_copy(nu_hbm.at[pl.ds(row0, RPT)], nuv)

    @pl.loop(0, CAP // LANES)
    def _fill(i):
        z = jnp.zeros((LANES,), jnp.int32)
        ctc[pl.ds(i * LANES, LANES)] = z
        utc[pl.ds(i * LANES, LANES)] = z

    plsc.subcore_barrier()

    def fire_idx(ci, bufs, sem):
        ctraw, utraw, ccv = bufs
        hoff = row0 * L + ci * IDX_N
        pltpu.async_copy(ct_hbm.at[pl.ds(hoff, IDX_N)], ctraw, sem)
        pltpu.async_copy(ut_hbm.at[pl.ds(hoff, IDX_N)], utraw, sem)
        pltpu.async_copy(cc_hbm.at[pl.ds(hoff, IDX_N)], ccv, sem)

    def wait_idx(ci, bufs, sem):
        ctraw, utraw, ccv = bufs
        hoff = row0 * L + ci * IDX_N
        pltpu.make_async_copy(ct_hbm.at[pl.ds(hoff, IDX_N)], ctraw, sem).wait()
        pltpu.make_async_copy(ut_hbm.at[pl.ds(hoff, IDX_N)], utraw, sem).wait()
        pltpu.make_async_copy(cc_hbm.at[pl.ds(hoff, IDX_N)], ccv, sem).wait()

    def process(ci, bufs):
        ctraw, utraw, ccv = bufs

        def row_compact(b, off):
            plsc.store_scatter(starts, [_splat(b)], off, mask=lane0)
            nusplat = plsc.load_gather(nuv, [_splat(ci * CB + b)])
            for g in range(13):
                l0 = g * 16
                base = b * L + l0
                lvec = lane + l0
                cc16 = ccv[pl.ds(base, LANES)]
                valid = (lvec < nusplat) & (cc16 == 0)
                if g == 12:
                    valid = valid & (lane < 8)
                cs = plsc.cumsum(valid.astype(jnp.int32))
                dst = off + cs - 1
                plsc.store_scatter(ctc, [dst], ctraw[pl.ds(base, LANES)],
                                   mask=valid)
                plsc.store_scatter(utc, [dst], utraw[pl.ds(base, LANES)],
                                   mask=valid)
                off = off + plsc.all_reduce_population_count(valid)
            odd = off & 1
            pad = lane0 & (odd == 1)
            plsc.store_scatter(ctc, [off], _splat(NC), mask=pad)
            plsc.store_scatter(utc, [off], _splat(0), mask=pad)
            return off + odd

        off = lax.fori_loop(0, CB, row_compact,
                            jnp.zeros((LANES,), jnp.int32))
        plsc.store_scatter(starts, [_splat(CB)], off, mask=lane0)
        total = _to_scalar(off)
        nwin = (total + (WIN - 1)) // WIN

        @pl.loop(0, nwin)
        def _fire(j):
            sl = pl.ds(j * WIN, WIN)
            pltpu.async_copy(spm_ct.at[ctc.at[sl]], ctrows.at[sl], semg)
            pltpu.async_copy(spm_ut.at[utc.at[sl]], utrows.at[sl], semg)

        @pl.loop(0, nwin)
        def _drain(j):
            sl = pl.ds(j * WIN, WIN)
            pltpu.make_async_copy(spm_ct.at[ctc.at[sl]],
                                  ctrows.at[sl], semg).wait()
            pltpu.make_async_copy(spm_ut.at[utc.at[sl]],
                                  utrows.at[sl], semg).wait()

        @pl.loop(0, CB)
        def _accum(b):
            sb = _to_scalar(plsc.load_gather(starts, [_splat(b)]))
            eb = _to_scalar(plsc.load_gather(starts, [_splat(b + 1)]))

            colw = lane & 7
            rowp = lane >> 3

            def pbody(i, accs):
                a0, a1 = accs
                ridx = _splat(2 * i) + rowp
                c16 = plsc.load_gather(ctrows, [ridx, colw])
                u16 = plsc.load_gather(utrows, [ridx, colw])
                c2 = plsc.bitcast(c16, jnp.bfloat16)
                u2 = plsc.bitcast(u16, jnp.bfloat16)
                pa, pb = plsc.unpack(c2 * u2,
                                     format=plsc.PackFormat.INTERLEAVED)
                return (a0 + pa, a1 + pb)

            z = jnp.zeros((LANES,), jnp.float32)
            a0, a1 = lax.fori_loop(sb // 2, eb // 2, pbody, (z, z))
            rot = (lane + 8) % 16
            ev = a0 + a0.at[rot].get(mode="promise_in_bounds")
            od = a1 + a1.at[rot].get(mode="promise_in_bounds")
            m8 = lane < 8
            plsc.store_scatter(outv, [_splat(b), 2 * lane], ev, mask=m8)
            plsc.store_scatter(outv, [_splat(b), 2 * lane + 1], od, mask=m8)

        orow = pl.ds(row0 + ci * CB, CB)

        @pl.when(c == 0)
        def _wlo():
            pltpu.sync_copy(outv, out_lo.at[orow])

        @pl.when(c == 1)
        def _whi():
            pltpu.sync_copy(outv, out_hi.at[orow])

    bufs0 = (ctraw0, utraw0, ccv0)
    bufs1 = (ctraw1, utraw1, ccv1)
    fire_idx(0, bufs0, semi0)

    @pl.loop(0, NCHUNK, step=2)
    def _chunks(ci):
        wait_idx(ci, bufs0, semi0)
        fire_idx(ci + 1, bufs1, semi1)
        process(ci, bufs0)
        wait_idx(ci + 1, bufs1, semi1)

        @pl.when(ci + 2 < NCHUNK)
        def _pf():
            fire_idx(ci + 2, bufs0, semi0)

        process(ci + 1, bufs1)


def kernel(ctype, ccont, utype, num_unit, ctype_w, utype_w, ccont_w):
    del ccont_w
    ct1 = ctype.reshape(B * L)
    ut1 = utype.reshape(B * L)
    cc1 = ccont.reshape(B * L)
    zpad = jnp.zeros((NPAD - NC, D), jnp.float32)
    ctw = jnp.concatenate([ctype_w, zpad]).astype(jnp.bfloat16)
    utw = jnp.concatenate([utype_w, zpad]).astype(jnp.bfloat16)

    def halves_i32(t):
        lo = lax.bitcast_convert_type(
            t[:, :DH].reshape(NPAD, DH // 2, 2), jnp.int32)
        hi = lax.bitcast_convert_type(
            t[:, DH:].reshape(NPAD, DH // 2, 2), jnp.int32)
        return lo, hi

    ctw_lo, ctw_hi = halves_i32(ctw)
    utw_lo, utw_hi = halves_i32(utw)

    mesh = plsc.VectorSubcoreMesh(
        core_axis_name="c", subcore_axis_name="s",
        num_cores=NCORES, num_subcores=NSUB)
    cp = pltpu.CompilerParams(
        needs_layout_passes=False, use_tc_tiling_on_sc=False)
    run = pl.kernel(
        _sc_body,
        out_type=(jax.ShapeDtypeStruct((B, DH), jnp.float32),
                  jax.ShapeDtypeStruct((B, DH), jnp.float32)),
        mesh=mesh,
        compiler_params=cp,
        scratch_types=[
            pltpu.VMEM((IDX_N,), jnp.int32),
            pltpu.VMEM((IDX_N,), jnp.int32),
            pltpu.VMEM((IDX_N,), jnp.int32),
            pltpu.VMEM((IDX_N,), jnp.int32),
            pltpu.VMEM((IDX_N,), jnp.int32),
            pltpu.VMEM((IDX_N,), jnp.int32),
            pltpu.VMEM((RPT,), jnp.int32),
            pltpu.VMEM((CAP,), jnp.int32),
            pltpu.VMEM((CAP,), jnp.int32),
            pltpu.VMEM((24,), jnp.int32),
            pltpu.VMEM((CAP, DH // 2), jnp.int32),
            pltpu.VMEM((CAP, DH // 2), jnp.int32),
            pltpu.VMEM((CB, DH), jnp.float32),
            pltpu.VMEM_SHARED((NPAD, DH // 2), jnp.int32),
            pltpu.VMEM_SHARED((NPAD, DH // 2), jnp.int32),
            pltpu.SemaphoreType.DMA,
            pltpu.SemaphoreType.DMA,
            pltpu.SemaphoreType.DMA,
        ],
    )
    lo, hi = run(ct1, cc1, ut1, num_unit, ctw_lo, ctw_hi, utw_lo, utw_hi)
    return jnp.concatenate([lo, hi], axis=1)
